# baseline (speedup 1.0000x reference)
import functools

import jax
import jax.numpy as jnp
from jax import lax
from jax.experimental import pallas as pl
from jax.experimental.pallas import tpu as pltpu
from jax.experimental.pallas import tpu_sc as plsc

NE = 16
EW = 512
TOPK = 2
M = 128
RB = 512
NC = 2
NS = 16
NW = NC * NS
L = 16


def _routing_body(x_ref, wr_ref, idx_ref, gate_ref):
    x = x_ref[...]
    wr = wr_ref[...]
    logits = jnp.dot(x, wr, preferred_element_type=jnp.float32)
    cols = lax.broadcasted_iota(jnp.int32, logits.shape, 1)
    neg = jnp.float32(-1e30)
    l0 = jnp.where(cols < NE, logits, neg)
    m1 = jnp.max(l0, axis=1, keepdims=True)
    a1 = jnp.min(jnp.where(l0 == m1, cols, 128), axis=1, keepdims=True)
    l1 = jnp.where(cols == a1, neg, l0)
    m2 = jnp.max(l1, axis=1, keepdims=True)
    a2 = jnp.min(jnp.where(l1 == m2, cols, 128), axis=1, keepdims=True)
    s1 = jax.nn.sigmoid(m1)
    s2 = jax.nn.sigmoid(m2)
    inv = 1.0 / (s1 + s2 + 1e-20)
    idx_ref[...] = jnp.where(cols == 0, a1, jnp.where(cols == 1, a2, 0))
    gate_ref[...] = jnp.where(cols == 0, s1 * inv,
                              jnp.where(cols == 1, s2 * inv, 0.0))


def _route(x_flat, w_router):
    T, D = x_flat.shape
    wr = jnp.zeros((D, 128), jnp.float32).at[:, :NE].set(w_router.T)
    idx, gate = pl.pallas_call(
        _routing_body,
        grid=(T // RB,),
        in_specs=[
            pl.BlockSpec((RB, D), lambda i: (i, 0)),
            pl.BlockSpec((D, 128), lambda i: (0, 0)),
        ],
        out_specs=[
            pl.BlockSpec((RB, 128), lambda i: (i, 0)),
            pl.BlockSpec((RB, 128), lambda i: (i, 0)),
        ],
        out_shape=[
            jax.ShapeDtypeStruct((T, 128), jnp.int32),
            jax.ShapeDtypeStruct((T, 128), jnp.float32),
        ],
    )(x_flat, wr)
    return idx, gate


def _lane():
    return lax.broadcasted_iota(jnp.int32, (L,), 0)


_GDN = lax.GatherDimensionNumbers(offset_dims=(), collapsed_slice_dims=(0,),
                                  start_index_map=(0,))


def _take(v, idx):
    return lax.gather(v, idx[:, None], _GDN, slice_sizes=(1,),
                      mode=lax.GatherScatterMode.PROMISE_IN_BOUNDS)


def _bcast_lane(v, j):
    return _take(v, jnp.full((L,), j, jnp.int32))


def _count_splat(mk):
    x = jnp.where(mk, 1, 0)
    lane = _lane()
    for sh in (1, 2, 4, 8):
        x = x + _take(x, lane ^ sh)
    return x


def _prefix_masked(mk):
    x = jnp.where(mk, 1, 0)
    lane = _lane()
    for sh in (1, 2, 4, 8):
        x = x + jnp.where(lane >= sh, _take(x, jnp.maximum(lane - sh, 0)), 0)
    return x


def _prefix_incl(x):
    lane = _lane()
    for sh in (1, 2, 4, 8):
        x = x + jnp.where(lane >= sh, _take(x, jnp.maximum(lane - sh, 0)), 0)
    return x


def _sc_counts(e_sm, cpw):
    mesh = plsc.VectorSubcoreMesh(core_axis_name="c", subcore_axis_name="s", num_cores=NC, num_subcores=NS)

    @functools.partial(
        pl.kernel,
        out_type=jax.ShapeDtypeStruct((NW, NE), jnp.int32),
        mesh=mesh,
        scratch_types=[pltpu.VMEM((cpw,), jnp.int32),
                       pltpu.VMEM((NE,), jnp.int32)],
    )
    def k(e_hbm, cnt_hbm, evec, cntv):
        wid = lax.axis_index("c") * NS + lax.axis_index("s")
        pltpu.sync_copy(e_hbm.at[pl.ds(wid * cpw, cpw)], evec)
        lane = _lane()

        def step(i, cnt):
            ev = evec[pl.ds(i * L, L)]
            for ex in range(NE):
                pc = _count_splat(ev == ex)
                cnt = cnt + jnp.where(lane == ex, pc, 0)
            return cnt

        counts = lax.fori_loop(0, cpw // L, step,
                               jnp.zeros((L,), jnp.int32))
        cntv[...] = counts
        pltpu.sync_copy(cntv, cnt_hbm.at[wid])

    return k(e_sm)


def _sc_dispatch(e_sm, x_flat, cnts, NP, nb, cpw):
    T, D = x_flat.shape
    rows_chunk = 16
    nchunks = cpw // rows_chunk
    mesh = plsc.VectorSubcoreMesh(core_axis_name="c", subcore_axis_name="s", num_cores=NC, num_subcores=NS)

    @functools.partial(
        pl.kernel,
        out_type=(
            jax.ShapeDtypeStruct((NP, D), jnp.float32),
            jax.ShapeDtypeStruct((TOPK * T // L, L), jnp.int32),
            jax.ShapeDtypeStruct((nb,), jnp.int32),
            jax.ShapeDtypeStruct((nb,), jnp.int32),
        ),
        mesh=mesh,
        scratch_types=[
            pltpu.VMEM((cpw,), jnp.int32),
            pltpu.VMEM((NW, NE), jnp.int32),
            pltpu.VMEM((cpw // L, L), jnp.int32),
            pltpu.VMEM((nb,), jnp.int32),
            pltpu.VMEM((nb,), jnp.int32),
            pltpu.VMEM((rows_chunk, D), jnp.float32),
            pltpu.VMEM((rows_chunk, D), jnp.float32),
            pltpu.SemaphoreType.DMA,
            pltpu.SemaphoreType.DMA,
            pltpu.SemaphoreType.DMA,
            pltpu.SemaphoreType.DMA,
        ],
    )
    def k(e_hbm, x_hbm, cnt_hbm, xs_hbm, posm_hbm, be_hbm, bv_hbm,
          evec, cnts_v, posm_v, be_v, bv_v, xbuf0, xbuf1,
          si0, si1, so0, so1, sem_unused=None):
        cid = lax.axis_index("c")
        sid = lax.axis_index("s")
        wid = cid * NS + sid
        lane = _lane()
        zero = jnp.zeros((L,), jnp.int32)

        pltpu.sync_copy(cnt_hbm, cnts_v)
        pltpu.sync_copy(e_hbm.at[pl.ds(wid * cpw, cpw)], evec)

        def red(i, carry):
            tot, pre = carry
            row = cnts_v[i]
            tot = tot + row
            pre = pre + jnp.where(i < wid, row, zero)
            return tot, pre

        tot, pre = lax.fori_loop(0, NW, red, (zero, zero))
        padded = lax.shift_left(
            lax.shift_right_logical(tot + (M - 1), M.bit_length() - 1),
            M.bit_length() - 1)
        pend = _prefix_incl(padded)
        poff = pend - padded
        basec = poff + pre

        ones = jnp.full((L,), 1, jnp.int32)

        def rank_step(i, basec):
            ev = evec[pl.ds(i * L, L)]
            pos = zero
            for ex in range(NE):
                mk = ev == ex
                csum = _prefix_masked(mk)
                pos = jnp.where(mk, _bcast_lane(basec, ex) + csum - 1, pos)
                basec = basec + jnp.where(lane == ex, _count_splat(mk), 0)
            posm_v[i] = pos
            return basec

        lax.fori_loop(0, cpw // L, rank_step, basec, unroll=False)

        pltpu.sync_copy(posm_v, posm_hbm.at[pl.ds(wid * (cpw // L), cpw // L)])

        tok_base = (wid % (T // cpw)) * cpw
        bufs = (xbuf0, xbuf1)
        sin = (si0, si1)
        sout = (so0, so1)

        def cp_in(c):
            return pltpu.async_copy(
                x_hbm.at[pl.ds(tok_base + c * rows_chunk, rows_chunk)],
                bufs[c % 2], sin[c % 2])

        hin = {0: cp_in(0)}
        if nchunks > 1:
            hin[1] = cp_in(1)
        hout = {}
        for c in range(nchunks):
            hin[c].wait()
            hout[c] = pltpu.async_copy(bufs[c % 2],
                                       xs_hbm.at[posm_v.at[c]], sout[c % 2])
            if c + 2 < nchunks:
                hout[c].wait()
                hin[c + 2] = cp_in(c + 2)
        for c in range(max(0, nchunks - 2), nchunks):
            hout[c].wait()

        gend = poff + tot
        @pl.when(wid == 0)
        def _():
            for cidx in range(nb // L):
                bstart = (lane + cidx * L) * M
                acc = zero
                for ex in range(NE):
                    acc = acc + jnp.where(_bcast_lane(pend, ex) <= bstart,
                                          ones, zero)
                bev = jnp.minimum(acc, NE - 1)
                be_v[pl.ds(cidx * L, L)] = bev
                bv_v[pl.ds(cidx * L, L)] = jnp.where(
                    bstart < _take(gend, bev), ones, zero)
            pltpu.sync_copy(be_v, be_hbm)
            pltpu.sync_copy(bv_v, bv_hbm)

    return k(e_sm, x_flat, cnts)


def _sc_combine(y, posm, g_sm, T, D):
    tpw = T // NW
    rows_chunk = 16
    nchunks = tpw // rows_chunk
    prow = tpw // L
    mesh = plsc.VectorSubcoreMesh(core_axis_name="c", subcore_axis_name="s",
                                  num_cores=NC, num_subcores=NS)

    @functools.partial(
        pl.kernel,
        out_type=jax.ShapeDtypeStruct((T, D), jnp.float32),
        mesh=mesh,
        scratch_types=[
            pltpu.VMEM((prow, L), jnp.int32),
            pltpu.VMEM((prow, L), jnp.int32),
            pltpu.VMEM((tpw,), jnp.float32),
            pltpu.VMEM((tpw,), jnp.float32),
            pltpu.VMEM((rows_chunk, D), jnp.float32),
            pltpu.VMEM((rows_chunk, D), jnp.float32),
            pltpu.VMEM((rows_chunk, D), jnp.float32),
            pltpu.VMEM((rows_chunk, D), jnp.float32),
            pltpu.SemaphoreType.DMA,
            pltpu.SemaphoreType.DMA,
        ],
    )
    def k(y_hbm, posm_hbm, g_hbm, out_hbm,
          p0m, p1m, g0v, g1v, a0, b0, a1, b1, s0, s1):
        wid = lax.axis_index("c") * NS + lax.axis_index("s")
        t0 = pl.multiple_of(wid * tpw, tpw)
        r0 = pl.multiple_of(wid * prow, prow)
        pltpu.sync_copy(posm_hbm.at[pl.ds(r0, prow)], p0m)
        pltpu.sync_copy(posm_hbm.at[pl.ds(T // L + r0, prow)], p1m)
        pltpu.sync_copy(g_hbm.at[pl.ds(t0, tpw)], g0v)
        pltpu.sync_copy(g_hbm.at[pl.ds(T + t0, tpw)], g1v)

        abufs = (a0, a1)
        bbufs = (b0, b1)
        sems = (s0, s1)

        def fetch(c):
            return (pltpu.async_copy(y_hbm.at[p0m.at[c]], abufs[c % 2],
                                     sems[c % 2]),
                    pltpu.async_copy(y_hbm.at[p1m.at[c]], bbufs[c % 2],
                                     sems[c % 2]))

        pending = {0: fetch(0)}
        for c in range(nchunks):
            if c + 1 < nchunks:
                pending[c + 1] = fetch(c + 1)
            ha, hb = pending[c]
            ha.wait()
            hb.wait()
            A = abufs[c % 2]
            Bb = bbufs[c % 2]
            g0c = g0v[pl.ds(c * L, L)]
            g1c = g1v[pl.ds(c * L, L)]

            def row(j, _, A=A, Bb=Bb, g0c=g0c, g1c=g1c):
                jj = jnp.full((L,), j, jnp.int32)
                g0 = _take(g0c, jj)
                g1 = _take(g1c, jj)
                for v in range(D // L):
                    sl = pl.ds(v * L, L)
                    A[j, sl] = A[j, sl] * g0 + Bb[j, sl] * g1
                return 0

            lax.fori_loop(0, rows_chunk, row, 0)
            pltpu.sync_copy(A, out_hbm.at[pl.ds(t0 + c * rows_chunk,
                                                rows_chunk)])

    return k(y, posm, g_sm)


def _gemm_body(be_ref, bv_ref, xs_ref, w1_ref, w2_ref, y_ref):
    del be_ref

    @pl.when(bv_ref[pl.program_id(0)] != 0)
    def _():
        x = xs_ref[...]
        h = jnp.dot(x, w1_ref[...], preferred_element_type=jnp.float32)
        a = jnp.square(jnp.maximum(h, 0.0))
        y_ref[...] = jnp.dot(a, w2_ref[...], preferred_element_type=jnp.float32)


def _grouped_gemm(xs, w1, w2, block_expert, block_valid):
    NP, D = xs.shape
    nb = NP // M
    grid_spec = pltpu.PrefetchScalarGridSpec(
        num_scalar_prefetch=2,
        grid=(nb,),
        in_specs=[
            pl.BlockSpec((M, D), lambda b, be, bv: (b, 0)),
            pl.BlockSpec((D, EW), lambda b, be, bv: (0, be[b])),
            pl.BlockSpec((EW, D), lambda b, be, bv: (be[b], 0)),
        ],
        out_specs=pl.BlockSpec((M, D), lambda b, be, bv: (b, 0)),
    )
    return pl.pallas_call(
        _gemm_body,
        grid_spec=grid_spec,
        out_shape=jax.ShapeDtypeStruct((NP, D), jnp.float32),
    )(block_expert, block_valid, xs, w1, w2)


def kernel(x, w_router, w1, w2):
    B, S, D = x.shape
    T = B * S
    P = TOPK * T
    cpw = P // NW
    NP = P + NE * M
    nb = NP // M
    x_flat = x.reshape(T, D)

    idx, gate = _route(x_flat, w_router)
    e_sm = jnp.concatenate([idx[:, 0], idx[:, 1]])

    cnts = _sc_counts(e_sm, cpw)
    xs, posm, be, bv = _sc_dispatch(e_sm, x_flat, cnts, NP, nb, cpw)
    y = _grouped_gemm(xs, w1, w2, be, bv)

    g_sm = jnp.concatenate([gate[:, 0], gate[:, 1]])
    out = _sc_combine(y, posm, g_sm, T, D)
    return out.reshape(B, S, D)


# speedup vs baseline: 1.1037x; 1.1037x over previous
import functools

import jax
import jax.numpy as jnp
from jax import lax
from jax.experimental import pallas as pl
from jax.experimental.pallas import tpu as pltpu
from jax.experimental.pallas import tpu_sc as plsc

NE = 16
EW = 512
TOPK = 2
M = 256
RB = 512
NC = 2
NS = 16
NW = NC * NS
L = 16


def _routing_body(x_ref, wr_ref, idx_ref, gate_ref):
    x = x_ref[...]
    wr = wr_ref[...]
    logits = jnp.dot(x, wr, preferred_element_type=jnp.float32)
    cols = lax.broadcasted_iota(jnp.int32, logits.shape, 1)
    neg = jnp.float32(-1e30)
    l0 = jnp.where(cols < NE, logits, neg)
    m1 = jnp.max(l0, axis=1, keepdims=True)
    a1 = jnp.min(jnp.where(l0 == m1, cols, 128), axis=1, keepdims=True)
    l1 = jnp.where(cols == a1, neg, l0)
    m2 = jnp.max(l1, axis=1, keepdims=True)
    a2 = jnp.min(jnp.where(l1 == m2, cols, 128), axis=1, keepdims=True)
    s1 = jax.nn.sigmoid(m1)
    s2 = jax.nn.sigmoid(m2)
    inv = 1.0 / (s1 + s2 + 1e-20)
    idx_ref[...] = jnp.where(cols == 0, a1, jnp.where(cols == 1, a2, 0))
    gate_ref[...] = jnp.where(cols == 0, s1 * inv,
                              jnp.where(cols == 1, s2 * inv, 0.0))


def _route(x_flat, w_router):
    T, D = x_flat.shape
    wr = jnp.zeros((D, 128), jnp.float32).at[:, :NE].set(w_router.T)
    idx, gate = pl.pallas_call(
        _routing_body,
        grid=(T // RB,),
        in_specs=[
            pl.BlockSpec((RB, D), lambda i: (i, 0)),
            pl.BlockSpec((D, 128), lambda i: (0, 0)),
        ],
        out_specs=[
            pl.BlockSpec((RB, 128), lambda i: (i, 0)),
            pl.BlockSpec((RB, 128), lambda i: (i, 0)),
        ],
        out_shape=[
            jax.ShapeDtypeStruct((T, 128), jnp.int32),
            jax.ShapeDtypeStruct((T, 128), jnp.float32),
        ],
    )(x_flat, wr)
    return idx, gate


def _lane():
    return lax.broadcasted_iota(jnp.int32, (L,), 0)


_GDN = lax.GatherDimensionNumbers(offset_dims=(), collapsed_slice_dims=(0,),
                                  start_index_map=(0,))


def _take(v, idx):
    return lax.gather(v, idx[:, None], _GDN, slice_sizes=(1,),
                      mode=lax.GatherScatterMode.PROMISE_IN_BOUNDS)


def _bcast_lane(v, j):
    return _take(v, jnp.full((L,), j, jnp.int32))


def _count_splat(mk):
    x = jnp.where(mk, 1, 0)
    lane = _lane()
    for sh in (1, 2, 4, 8):
        x = x + _take(x, lane ^ sh)
    return x


def _prefix_masked(mk):
    x = jnp.where(mk, 1, 0)
    lane = _lane()
    for sh in (1, 2, 4, 8):
        x = x + jnp.where(lane >= sh, _take(x, jnp.maximum(lane - sh, 0)), 0)
    return x


def _prefix_incl(x):
    lane = _lane()
    for sh in (1, 2, 4, 8):
        x = x + jnp.where(lane >= sh, _take(x, jnp.maximum(lane - sh, 0)), 0)
    return x


def _sc_counts(e_sm, cpw):
    mesh = plsc.VectorSubcoreMesh(core_axis_name="c", subcore_axis_name="s", num_cores=NC, num_subcores=NS)

    @functools.partial(
        pl.kernel,
        out_type=jax.ShapeDtypeStruct((NW, NE), jnp.int32),
        mesh=mesh,
        scratch_types=[pltpu.VMEM((cpw,), jnp.int32),
                       pltpu.VMEM((NE,), jnp.int32)],
    )
    def k(e_hbm, cnt_hbm, evec, cntv):
        wid = lax.axis_index("c") * NS + lax.axis_index("s")
        pltpu.sync_copy(e_hbm.at[pl.ds(wid * cpw, cpw)], evec)
        lane = _lane()

        def step(i, cnt):
            ev = evec[pl.ds(i * L, L)]
            for ex in range(NE):
                pc = _count_splat(ev == ex)
                cnt = cnt + jnp.where(lane == ex, pc, 0)
            return cnt

        counts = lax.fori_loop(0, cpw // L, step,
                               jnp.zeros((L,), jnp.int32))
        cntv[...] = counts
        pltpu.sync_copy(cntv, cnt_hbm.at[wid])

    return k(e_sm)


def _sc_dispatch(e_sm, x_flat, cnts, NP, nb, cpw):
    T, D = x_flat.shape
    rows_chunk = 32
    nchunks = cpw // rows_chunk
    mesh = plsc.VectorSubcoreMesh(core_axis_name="c", subcore_axis_name="s", num_cores=NC, num_subcores=NS)

    @functools.partial(
        pl.kernel,
        out_type=(
            jax.ShapeDtypeStruct((NP, D), jnp.float32),
            jax.ShapeDtypeStruct((TOPK * T // L, L), jnp.int32),
            jax.ShapeDtypeStruct((nb,), jnp.int32),
            jax.ShapeDtypeStruct((nb,), jnp.int32),
        ),
        mesh=mesh,
        scratch_types=[
            pltpu.VMEM((cpw,), jnp.int32),
            pltpu.VMEM((NW, NE), jnp.int32),
            pltpu.VMEM((cpw // L, L), jnp.int32),
            pltpu.VMEM((nb,), jnp.int32),
            pltpu.VMEM((nb,), jnp.int32),
            pltpu.VMEM((rows_chunk, D), jnp.float32),
            pltpu.VMEM((rows_chunk, D), jnp.float32),
            pltpu.SemaphoreType.DMA,
            pltpu.SemaphoreType.DMA,
            pltpu.SemaphoreType.DMA,
            pltpu.SemaphoreType.DMA,
        ],
    )
    def k(e_hbm, x_hbm, cnt_hbm, xs_hbm, posm_hbm, be_hbm, bv_hbm,
          evec, cnts_v, posm_v, be_v, bv_v, xbuf0, xbuf1,
          si0, si1, so0, so1, sem_unused=None):
        cid = lax.axis_index("c")
        sid = lax.axis_index("s")
        wid = cid * NS + sid
        lane = _lane()
        zero = jnp.zeros((L,), jnp.int32)

        pltpu.sync_copy(cnt_hbm, cnts_v)
        pltpu.sync_copy(e_hbm.at[pl.ds(wid * cpw, cpw)], evec)

        def red(i, carry):
            tot, pre = carry
            row = cnts_v[i]
            tot = tot + row
            pre = pre + jnp.where(i < wid, row, zero)
            return tot, pre

        tot, pre = lax.fori_loop(0, NW, red, (zero, zero))
        padded = lax.shift_left(
            lax.shift_right_logical(tot + (M - 1), M.bit_length() - 1),
            M.bit_length() - 1)
        pend = _prefix_incl(padded)
        poff = pend - padded
        basec = poff + pre

        ones = jnp.full((L,), 1, jnp.int32)

        tok_base = (wid % (T // cpw)) * cpw
        bufs = (xbuf0, xbuf1)
        sin = (si0, si1)
        sout = (so0, so1)
        hin = {}

        def rank_step(i, basec):
            ev = evec[pl.ds(i * L, L)]
            pos = zero
            for ex in range(NE):
                mk = ev == ex
                csum = _prefix_masked(mk)
                pos = jnp.where(mk, _bcast_lane(basec, ex) + csum - 1, pos)
                basec = basec + jnp.where(lane == ex, _count_splat(mk), 0)
            posm_v[i] = pos
            return basec

        hin[0] = pltpu.async_copy(
            x_hbm.at[pl.ds(tok_base, rows_chunk)], bufs[0], sin[0])
        if nchunks > 1:
            hin[1] = pltpu.async_copy(
                x_hbm.at[pl.ds(tok_base + rows_chunk, rows_chunk)],
                bufs[1], sin[1])
        lax.fori_loop(0, cpw // L, rank_step, basec, unroll=False)

        pltpu.sync_copy(posm_v, posm_hbm.at[pl.ds(wid * (cpw // L), cpw // L)])

        def cp_in(c):
            return pltpu.async_copy(
                x_hbm.at[pl.ds(tok_base + c * rows_chunk, rows_chunk)],
                bufs[c % 2], sin[c % 2])

        hout = {}
        for c in range(nchunks):
            hin[c].wait()
            h0 = pltpu.async_copy(bufs[c % 2].at[pl.ds(0, L)],
                                  xs_hbm.at[posm_v.at[2 * c]], sout[c % 2])
            h1 = pltpu.async_copy(bufs[c % 2].at[pl.ds(L, L)],
                                  xs_hbm.at[posm_v.at[2 * c + 1]], sout[c % 2])
            hout[c] = (h0, h1)
            if c + 2 < nchunks:
                h0.wait()
                h1.wait()
                hin[c + 2] = cp_in(c + 2)
        for c in range(max(0, nchunks - 2), nchunks):
            hout[c][0].wait()
            hout[c][1].wait()

        gend = poff + tot
        @pl.when(wid == 0)
        def _():
            for cidx in range(nb // L):
                bstart = (lane + cidx * L) * M
                acc = zero
                for ex in range(NE):
                    acc = acc + jnp.where(_bcast_lane(pend, ex) <= bstart,
                                          ones, zero)
                bev = jnp.minimum(acc, NE - 1)
                be_v[pl.ds(cidx * L, L)] = bev
                bv_v[pl.ds(cidx * L, L)] = jnp.where(
                    bstart < _take(gend, bev), ones, zero)
            pltpu.sync_copy(be_v, be_hbm)
            pltpu.sync_copy(bv_v, bv_hbm)

    return k(e_sm, x_flat, cnts)


def _sc_combine(y, posm, g_sm, T, D):
    tpw = T // NW
    rows_chunk = 16
    nchunks = tpw // rows_chunk
    prow = tpw // L
    mesh = plsc.VectorSubcoreMesh(core_axis_name="c", subcore_axis_name="s",
                                  num_cores=NC, num_subcores=NS)

    @functools.partial(
        pl.kernel,
        out_type=jax.ShapeDtypeStruct((T, D), jnp.float32),
        mesh=mesh,
        scratch_types=[
            pltpu.VMEM((prow, L), jnp.int32),
            pltpu.VMEM((prow, L), jnp.int32),
            pltpu.VMEM((tpw,), jnp.float32),
            pltpu.VMEM((tpw,), jnp.float32),
            pltpu.VMEM((rows_chunk, D), jnp.float32),
            pltpu.VMEM((rows_chunk, D), jnp.float32),
            pltpu.VMEM((rows_chunk, D), jnp.float32),
            pltpu.VMEM((rows_chunk, D), jnp.float32),
            pltpu.SemaphoreType.DMA,
            pltpu.SemaphoreType.DMA,
        ],
    )
    def k(y_hbm, posm_hbm, g_hbm, out_hbm,
          p0m, p1m, g0v, g1v, a0, b0, a1, b1, s0, s1):
        wid = lax.axis_index("c") * NS + lax.axis_index("s")
        t0 = pl.multiple_of(wid * tpw, tpw)
        r0 = pl.multiple_of(wid * prow, prow)
        pltpu.sync_copy(posm_hbm.at[pl.ds(r0, prow)], p0m)
        pltpu.sync_copy(posm_hbm.at[pl.ds(T // L + r0, prow)], p1m)
        pltpu.sync_copy(g_hbm.at[pl.ds(t0, tpw)], g0v)
        pltpu.sync_copy(g_hbm.at[pl.ds(T + t0, tpw)], g1v)

        abufs = (a0, a1)
        bbufs = (b0, b1)
        sems = (s0, s1)

        def fetch(c):
            return (pltpu.async_copy(y_hbm.at[p0m.at[c]], abufs[c % 2],
                                     sems[c % 2]),
                    pltpu.async_copy(y_hbm.at[p1m.at[c]], bbufs[c % 2],
                                     sems[c % 2]))

        pending = {0: fetch(0)}
        for c in range(nchunks):
            if c + 1 < nchunks:
                pending[c + 1] = fetch(c + 1)
            ha, hb = pending[c]
            ha.wait()
            hb.wait()
            A = abufs[c % 2]
            Bb = bbufs[c % 2]
            g0c = g0v[pl.ds(c * L, L)]
            g1c = g1v[pl.ds(c * L, L)]

            def row(j, _, A=A, Bb=Bb, g0c=g0c, g1c=g1c):
                jj = jnp.full((L,), j, jnp.int32)
                g0 = _take(g0c, jj)
                g1 = _take(g1c, jj)
                for v in range(D // L):
                    sl = pl.ds(v * L, L)
                    A[j, sl] = A[j, sl] * g0 + Bb[j, sl] * g1
                return 0

            lax.fori_loop(0, rows_chunk, row, 0)
            pltpu.sync_copy(A, out_hbm.at[pl.ds(t0 + c * rows_chunk,
                                                rows_chunk)])

    return k(y, posm, g_sm)


def _gemm_body(be_ref, bv_ref, xs_ref, w1_ref, w2_ref, y_ref):
    del be_ref

    @pl.when(bv_ref[pl.program_id(0)] != 0)
    def _():
        x = xs_ref[...]
        h = jnp.dot(x, w1_ref[...], preferred_element_type=jnp.float32)
        a = jnp.square(jnp.maximum(h, 0.0))
        y_ref[...] = jnp.dot(a, w2_ref[...], preferred_element_type=jnp.float32)


def _grouped_gemm(xs, w1, w2, block_expert, block_valid):
    NP, D = xs.shape
    nb = NP // M
    grid_spec = pltpu.PrefetchScalarGridSpec(
        num_scalar_prefetch=2,
        grid=(nb,),
        in_specs=[
            pl.BlockSpec((M, D), lambda b, be, bv: (b, 0)),
            pl.BlockSpec((D, EW), lambda b, be, bv: (0, be[b])),
            pl.BlockSpec((EW, D), lambda b, be, bv: (be[b], 0)),
        ],
        out_specs=pl.BlockSpec((M, D), lambda b, be, bv: (b, 0)),
    )
    return pl.pallas_call(
        _gemm_body,
        grid_spec=grid_spec,
        out_shape=jax.ShapeDtypeStruct((NP, D), jnp.float32),
    )(block_expert, block_valid, xs, w1, w2)


def kernel(x, w_router, w1, w2):
    B, S, D = x.shape
    T = B * S
    P = TOPK * T
    cpw = P // NW
    NP = P + NE * M
    nb = NP // M
    x_flat = x.reshape(T, D)

    idx, gate = _route(x_flat, w_router)
    e_sm = jnp.concatenate([idx[:, 0], idx[:, 1]])

    cnts = _sc_counts(e_sm, cpw)
    xs, posm, be, bv = _sc_dispatch(e_sm, x_flat, cnts, NP, nb, cpw)
    y = _grouped_gemm(xs, w1, w2, be, bv)

    g_sm = jnp.concatenate([gate[:, 0], gate[:, 1]])
    out = _sc_combine(y, posm, g_sm, T, D)
    return out.reshape(B, S, D)
